# Initial kernel scaffold; baseline (speedup 1.0000x reference)
import jax
import jax.numpy as jnp
from jax.experimental import pallas as pl


def kernel(residue_name_one_hot, atom_type_one_hot, record_symbol_one_hot, rdkit_atom_feature_onehot, edge_index, bond_feature, batch, b_factor, W_edge, b_edge, W_node, b_node, W_out, b_out):
    raise NotImplementedError("write your pallas kernel here")


# speedup vs baseline: 2.7960x; 2.7960x over previous
import functools

import jax
import jax.numpy as jnp
from jax import lax
from jax.experimental import pallas as pl
from jax.experimental.pallas import tpu as pltpu
from jax.experimental.pallas import tpu_sc as plsc

N_NODES = 10000
N_EDGES = 320000
D_NODE = 142
D_EDGE = 5
D_HID = 128
LANES = 16

NC = 2
NS = 16
NW = NC * NS
E_PER_W = N_EDGES // NW
CHUNK = 80
N_CHUNKS = E_PER_W // CHUNK
N_UNITS = N_NODES // CHUNK
UNITS_PER_TILE = -(-N_UNITS // NS)


def _node_mm_body(r_ref, a_ref, c_ref, d_ref,
                  we0, we1, we2, we3,
                  wn0, wn1, wn2, wn3, bn_ref,
                  h_ref, xn_ref):
    r = r_ref[...]
    a = a_ref[...]
    c = c_ref[...]
    d = d_ref[...]
    dot = functools.partial(jnp.dot, preferred_element_type=jnp.float32)
    h_ref[...] = (dot(r, we0[...]) + dot(a, we1[...])
                  + dot(c, we2[...]) + dot(d, we3[...]))
    xn_ref[...] = (dot(r, wn0[...]) + dot(a, wn1[...])
                   + dot(c, wn2[...]) + dot(d, wn3[...]) + bn_ref[...])


def _node_matmuls(r, a, c, d, wes, wns, b_node):
    return pl.pallas_call(
        _node_mm_body,
        out_shape=(
            jax.ShapeDtypeStruct((N_NODES, D_HID), jnp.float32),
            jax.ShapeDtypeStruct((N_NODES, D_HID), jnp.float32),
        ),
    )(r, a, c, d, *wes, *wns, b_node)


_G_BLK = 8000


def _edge_bias_body(bond_ref, web_ref, be_ref, g_ref):
    g_ref[...] = (jnp.dot(bond_ref[...], web_ref[...],
                          preferred_element_type=jnp.float32) + be_ref[...])


def _edge_bias(bond, W_eb, b_edge):
    grid = (N_EDGES // _G_BLK,)
    return pl.pallas_call(
        _edge_bias_body,
        grid=grid,
        in_specs=[
            pl.BlockSpec((_G_BLK, D_EDGE), lambda i: (i, 0)),
            pl.BlockSpec((D_EDGE, D_HID), lambda i: (0, 0)),
            pl.BlockSpec((1, D_HID), lambda i: (0, 0)),
        ],
        out_specs=pl.BlockSpec((_G_BLK, D_HID), lambda i: (i, 0)),
        out_shape=jax.ShapeDtypeStruct((N_EDGES, D_HID), jnp.float32),
    )(bond, W_eb, b_edge)


_sc_mesh = plsc.VectorSubcoreMesh(core_axis_name="c", subcore_axis_name="s")


@functools.partial(
    pl.kernel,
    mesh=_sc_mesh,
    out_type=jax.ShapeDtypeStruct((NC, N_NODES, D_HID), jnp.float32),
    scratch_types=[
        pltpu.VMEM((CHUNK,), jnp.int32),
        pltpu.VMEM((CHUNK,), jnp.int32),
        pltpu.VMEM((CHUNK, D_HID), jnp.float32),
        pltpu.VMEM((CHUNK, D_HID), jnp.float32),
        pltpu.VMEM_SHARED((N_NODES, D_HID), jnp.float32),
        pltpu.SemaphoreType.DMA,
    ],
)
def _sc_edge_kernel(h_hbm, g_hbm, src_hbm, dst_hbm, out_hbm,
                    src_v, dst_v, rows_v, g_v, agg_sh, sem):
    c = lax.axis_index("c")
    s = lax.axis_index("s")
    wid = s * NC + c

    zero = jnp.zeros((LANES,), jnp.float32)

    def zrow(r, carry):
        for k in range(D_HID // LANES):
            rows_v[r, pl.ds(k * LANES, LANES)] = zero
        return carry

    lax.fori_loop(0, CHUNK, zrow, 0)
    for t in range(UNITS_PER_TILE):
        u = s * UNITS_PER_TILE + t

        @pl.when(u < N_UNITS)
        def _():
            pltpu.sync_copy(rows_v, agg_sh.at[pl.ds(u * CHUNK, CHUNK)])

    plsc.subcore_barrier()

    def chunk_body(ci, carry):
        base = wid * E_PER_W + ci * CHUNK
        pltpu.sync_copy(src_hbm.at[wid, ci], src_v)
        pltpu.sync_copy(dst_hbm.at[wid, ci], dst_v)
        pltpu.sync_copy(g_hbm.at[pl.ds(base, CHUNK)], g_v)
        pltpu.async_copy(h_hbm.at[src_v], rows_v, sem).wait()

        def row_body(r, inner):
            for k in range(D_HID // LANES):
                sl = pl.ds(k * LANES, LANES)
                rows_v[r, sl] = jnp.maximum(rows_v[r, sl] + g_v[r, sl], 0.0)
            return inner

        lax.fori_loop(0, CHUNK, row_body, 0)
        pltpu.sync_copy(rows_v, agg_sh.at[dst_v], add=True)
        return carry

    lax.fori_loop(0, N_CHUNKS, chunk_body, 0)
    plsc.subcore_barrier()

    for t in range(UNITS_PER_TILE):
        u = s * UNITS_PER_TILE + t

        @pl.when(u < N_UNITS)
        def _():
            pltpu.sync_copy(agg_sh.at[pl.ds(u * CHUNK, CHUNK)], rows_v)
            pltpu.sync_copy(rows_v, out_hbm.at[c, pl.ds(u * CHUNK, CHUNK)])


def _head_body(xn_ref, agg_ref, wout_ref, bout_ref, out_ref):
    emb = jnp.maximum(xn_ref[...] + agg_ref[0] + agg_ref[1], 0.0)
    logits = (jnp.dot(emb, wout_ref[...], preferred_element_type=jnp.float32)
              + bout_ref[...])
    out_ref[...] = jax.nn.sigmoid(logits)


def _head(xn, agg2, W_out, b_out):
    return pl.pallas_call(
        _head_body,
        out_shape=jax.ShapeDtypeStruct((N_NODES, 1), jnp.float32),
    )(xn, agg2, W_out, b_out)


def kernel(residue_name_one_hot, atom_type_one_hot, record_symbol_one_hot,
           rdkit_atom_feature_onehot, edge_index, bond_feature, batch, b_factor,
           W_edge, b_edge, W_node, b_node, W_out, b_out):
    del batch
    splits = (21, 38, 3, 80)
    offs = (0, 21, 59, 62)
    wes = tuple(lax.slice_in_dim(W_edge, o, o + sz, axis=0)
                for o, sz in zip(offs, splits))
    wns = tuple(lax.slice_in_dim(W_node, o, o + sz, axis=0)
                for o, sz in zip(offs, splits))
    W_eb = lax.slice_in_dim(W_edge, D_NODE, D_NODE + D_EDGE, axis=0)

    h, xn = _node_matmuls(residue_name_one_hot, atom_type_one_hot,
                          record_symbol_one_hot, rdkit_atom_feature_onehot,
                          wes, wns, b_node.reshape(1, D_HID))
    g = _edge_bias(bond_feature, W_eb, b_edge.reshape(1, D_HID))

    src2 = edge_index[0].reshape(NW, N_CHUNKS, CHUNK)
    dst2 = edge_index[1].reshape(NW, N_CHUNKS, CHUNK)
    agg2 = _sc_edge_kernel(h, g, src2, dst2)

    out = _head(xn, agg2, W_out, b_out.reshape(1, 1))
    return out.reshape(N_NODES), b_factor
